# baseline (speedup 1.0000x reference)
import functools

import jax
import jax.numpy as jnp
from jax import lax
from jax.experimental import pallas as pl
from jax.experimental.pallas import tpu as pltpu
from jax.experimental.pallas import tpu_sc as plsc

NC = 2
NS = 16
NW = NC * NS
LANES = 16
CH = 128


def _mesh():
    return plsc.VectorSubcoreMesh(
        core_axis_name="c", subcore_axis_name="s", num_cores=NC, num_subcores=NS
    )


def _make_deg_y(n_pad, e_pad, d):
    rows_per_sub = n_pad // NS
    rows_per_tile = n_pad // NW
    n_chunks_tot = e_pad // CH
    n_chunks = n_chunks_tot // NS

    @functools.partial(
        pl.kernel,
        out_type=[
            jax.ShapeDtypeStruct((n_pad, d), jnp.float32),
            jax.ShapeDtypeStruct((n_pad,), jnp.float32),
        ],
        mesh=_mesh(),
        scratch_types=[
            pltpu.VMEM_SHARED((n_pad,), jnp.float32),
            pltpu.VMEM((rows_per_sub,), jnp.float32),
            pltpu.VMEM((rows_per_sub,), jnp.float32),
            pltpu.VMEM((CH,), jnp.float32),
            pltpu.VMEM((n_chunks, CH), jnp.int32),
            pltpu.VMEM((rows_per_tile, d), jnp.float32),
        ],
    )
    def deg_y(rows_hbm, x_hbm, y_hbm, inv_hbm, deg_sh, buf_a, buf_b, ones_v,
              idx_v, xblk):
        c = lax.axis_index("c")
        s = lax.axis_index("s")

        pltpu.sync_copy(rows_hbm.at[pl.ds(s * n_chunks, n_chunks)], idx_v)

        def zero_loop(i, carry):
            buf_a[pl.ds(i * LANES, LANES)] = jnp.zeros((LANES,), jnp.float32)
            return carry

        lax.fori_loop(0, rows_per_sub // LANES, zero_loop, None)
        pltpu.sync_copy(buf_a, deg_sh.at[pl.ds(s * rows_per_sub, rows_per_sub)])

        def ones_loop(i, carry):
            ones_v[pl.ds(i * LANES, LANES)] = jnp.ones((LANES,), jnp.float32)
            return carry

        lax.fori_loop(0, CH // LANES, ones_loop, None)
        plsc.subcore_barrier()

        def acc(k, carry):
            pltpu.sync_copy(ones_v, deg_sh.at[idx_v.at[k]], add=True)
            return carry

        lax.fori_loop(0, n_chunks, acc, None)
        plsc.subcore_barrier()

        pltpu.sync_copy(deg_sh.at[pl.ds(s * rows_per_sub, rows_per_sub)], buf_a)

        def inv_loop(i, carry):
            v = buf_a[pl.ds(i * LANES, LANES)]
            buf_b[pl.ds(i * LANES, LANES)] = 1.0 / (v + 1.0)
            return carry

        lax.fori_loop(0, rows_per_sub // LANES, inv_loop, None)

        @pl.when(c == 0)
        def _():
            pltpu.sync_copy(buf_b, inv_hbm.at[pl.ds(s * rows_per_sub, rows_per_sub)])

        w = s * NC + c
        rowbase = w * rows_per_tile
        pltpu.sync_copy(x_hbm.at[pl.ds(rowbase, rows_per_tile)], xblk)

        def y_loop(g, carry):
            inv16 = buf_b[pl.ds(c * rows_per_tile + g * LANES, LANES)]
            for j in range(LANES):
                iv = inv16[j]
                r = g * LANES + j
                for cc in range(d // LANES):
                    xv = xblk[r, pl.ds(cc * LANES, LANES)]
                    xblk[r, pl.ds(cc * LANES, LANES)] = xv * iv
            return carry

        lax.fori_loop(0, rows_per_tile // LANES, y_loop, None)
        pltpu.sync_copy(xblk, y_hbm.at[pl.ds(rowbase, rows_per_tile)])

    return deg_y


def _make_spmm(n_pad, e_pad, d):
    rows_per_sub = n_pad // NS
    chunks_per_tile = e_pad // CH // NW
    n_pairs = chunks_per_tile // 2
    zrows = 16

    @functools.partial(
        pl.kernel,
        out_type=jax.ShapeDtypeStruct((NC * n_pad, d), jnp.float32),
        mesh=_mesh(),
        scratch_types=[
            pltpu.VMEM_SHARED((n_pad, d), jnp.float32),
            pltpu.VMEM((zrows, d), jnp.float32),
            pltpu.VMEM((chunks_per_tile, CH), jnp.int32),
            pltpu.VMEM((chunks_per_tile, CH), jnp.int32),
            pltpu.VMEM((CH, d), jnp.float32),
            pltpu.SemaphoreType.DMA,
        ],
    )
    def spmm(rows_hbm, cols_hbm, y_hbm, p_hbm, s_sh, zb, ridx, cidx, g0, sem0):
        c = lax.axis_index("c")
        s = lax.axis_index("s")
        w = s * NC + c
        cbase = w * chunks_per_tile

        pltpu.sync_copy(rows_hbm.at[pl.ds(cbase, chunks_per_tile)], ridx)
        pltpu.sync_copy(cols_hbm.at[pl.ds(cbase, chunks_per_tile)], cidx)

        pltpu.async_copy(y_hbm.at[cidx.at[0]], g0, sem0)

        for r in range(zrows):
            for cc in range(d // LANES):
                zb[r, pl.ds(cc * LANES, LANES)] = jnp.zeros((LANES,), jnp.float32)

        def zero_loop(i, carry):
            pltpu.sync_copy(zb, s_sh.at[pl.ds(s * rows_per_sub + i * zrows, zrows)])
            return carry

        lax.fori_loop(0, rows_per_sub // zrows, zero_loop, None)
        plsc.subcore_barrier()

        def chunk(k, carry):
            pltpu.make_async_copy(y_hbm.at[cidx.at[k]], g0, sem0).wait()
            pltpu.sync_copy(g0, s_sh.at[ridx.at[k]], add=True)

            @pl.when(k < chunks_per_tile - 1)
            def _():
                pltpu.async_copy(y_hbm.at[cidx.at[k + 1]], g0, sem0)

            return carry

        lax.fori_loop(0, chunks_per_tile, chunk, None)
        plsc.subcore_barrier()

        pltpu.sync_copy(
            s_sh.at[pl.ds(s * rows_per_sub, rows_per_sub)],
            p_hbm.at[pl.ds(c * n_pad + s * rows_per_sub, rows_per_sub)],
        )

    return spmm


def _combine_body(x_ref, p0_ref, p1_ref, inv_ref, w_ref, b_ref, o_ref):
    inv = inv_ref[...]
    support = (p0_ref[...] + p1_ref[...]) * inv + (inv * inv) * x_ref[...]
    o_ref[...] = (
        lax.dot_general(
            support, w_ref[...], (((1,), (1,)), ((), ())),
            preferred_element_type=jnp.float32,
        )
        + b_ref[...]
    )


def _combine(x_pad, p0, p1, inv, W, b2, blk):
    n_pad, d = x_pad.shape
    grid = (n_pad // blk,)
    bs = pl.BlockSpec((blk, d), lambda i: (i, 0))
    return pl.pallas_call(
        _combine_body,
        grid=grid,
        in_specs=[
            bs, bs, bs,
            pl.BlockSpec((blk, 1), lambda i: (i, 0)),
            pl.BlockSpec((d, d), lambda i: (0, 0)),
            pl.BlockSpec((1, d), lambda i: (0, 0)),
        ],
        out_specs=bs,
        out_shape=jax.ShapeDtypeStruct((n_pad, d), jnp.float32),
    )(x_pad, p0, p1, inv, W, b2)


def kernel(x, edge_index, W, b):
    n, d = x.shape
    e = edge_index.shape[1]
    n_pad = ((n + 8 * NW - 1) // (8 * NW)) * (8 * NW)
    ept = -(-e // NW)
    ept = -(-ept // (2 * CH)) * (2 * CH)
    e_pad = ept * NW

    rows = edge_index[0].astype(jnp.int32)
    cols = edge_index[1].astype(jnp.int32)
    rows_p = jnp.pad(rows, (0, e_pad - e), constant_values=n)
    cols_p = jnp.pad(cols, (0, e_pad - e), constant_values=n)
    x_pad = jnp.pad(x, ((0, n_pad - n), (0, 0)))

    y_pad, inv = _make_deg_y(n_pad, e_pad, d)(rows_p.reshape(-1, CH), x_pad)
    p = _make_spmm(n_pad, e_pad, d)(
        rows_p.reshape(-1, CH), cols_p.reshape(-1, CH), y_pad
    )
    out_pad = _combine(
        x_pad, p[:n_pad], p[n_pad:], inv.reshape(-1, 1), W, b.reshape(1, -1),
        blk=1024,
    )
    return out_pad[:n]


# speedup vs baseline: 1.3646x; 1.3646x over previous
import functools

import jax
import jax.numpy as jnp
from jax import lax
from jax.experimental import pallas as pl
from jax.experimental.pallas import tpu as pltpu
from jax.experimental.pallas import tpu_sc as plsc

NC = 2
NS = 16
NW = NC * NS
LANES = 16
CH = 80


def _mesh():
    return plsc.VectorSubcoreMesh(
        core_axis_name="c", subcore_axis_name="s", num_cores=NC, num_subcores=NS
    )


def _make_deg_y(n_pad, e, d):
    rows_per_sub = n_pad // NS
    rows_per_tile = n_pad // NW
    edges_per_sub = e // NS
    n_chunks = edges_per_sub // CH

    @functools.partial(
        pl.kernel,
        out_type=[
            jax.ShapeDtypeStruct((n_pad, d), jnp.float32),
            jax.ShapeDtypeStruct((n_pad,), jnp.float32),
        ],
        mesh=_mesh(),
        scratch_types=[
            pltpu.VMEM_SHARED((n_pad,), jnp.float32),
            pltpu.VMEM((rows_per_sub,), jnp.float32),
            pltpu.VMEM((rows_per_sub,), jnp.float32),
            pltpu.VMEM((CH,), jnp.float32),
            pltpu.VMEM((CH,), jnp.int32),
            pltpu.VMEM((rows_per_tile, d), jnp.float32),
        ],
    )
    def deg_y(rows_hbm, x_hbm, y_hbm, inv_hbm, deg_sh, buf_a, buf_b, ones_v,
              idx_v, xblk):
        c = lax.axis_index("c")
        s = lax.axis_index("s")

        def zero_loop(i, carry):
            buf_a[pl.ds(i * LANES, LANES)] = jnp.zeros((LANES,), jnp.float32)
            return carry

        lax.fori_loop(0, rows_per_sub // LANES, zero_loop, None)
        pltpu.sync_copy(buf_a, deg_sh.at[pl.ds(s * rows_per_sub, rows_per_sub)])

        def ones_loop(i, carry):
            ones_v[pl.ds(i * LANES, LANES)] = jnp.ones((LANES,), jnp.float32)
            return carry

        lax.fori_loop(0, CH // LANES, ones_loop, None)
        plsc.subcore_barrier()

        def acc(k, carry):
            base = s * edges_per_sub + k * CH
            pltpu.sync_copy(rows_hbm.at[pl.ds(base, CH)], idx_v)
            pltpu.sync_copy(ones_v, deg_sh.at[idx_v], add=True)
            return carry

        lax.fori_loop(0, n_chunks, acc, None)
        plsc.subcore_barrier()

        pltpu.sync_copy(deg_sh.at[pl.ds(s * rows_per_sub, rows_per_sub)], buf_a)

        def inv_loop(i, carry):
            v = buf_a[pl.ds(i * LANES, LANES)]
            buf_b[pl.ds(i * LANES, LANES)] = 1.0 / (v + 1.0)
            return carry

        lax.fori_loop(0, rows_per_sub // LANES, inv_loop, None)

        @pl.when(c == 0)
        def _():
            pltpu.sync_copy(buf_b, inv_hbm.at[pl.ds(s * rows_per_sub, rows_per_sub)])

        w = s * NC + c
        rowbase = w * rows_per_tile
        pltpu.sync_copy(x_hbm.at[pl.ds(rowbase, rows_per_tile)], xblk)

        def y_loop(g, carry):
            inv16 = buf_b[pl.ds(c * rows_per_tile + g * LANES, LANES)]
            for j in range(LANES):
                iv = inv16[j]
                r = g * LANES + j
                for cc in range(d // LANES):
                    xv = xblk[r, pl.ds(cc * LANES, LANES)]
                    xblk[r, pl.ds(cc * LANES, LANES)] = xv * iv
            return carry

        lax.fori_loop(0, rows_per_tile // LANES, y_loop, None)
        pltpu.sync_copy(xblk, y_hbm.at[pl.ds(rowbase, rows_per_tile)])

    return deg_y


def _make_spmm(n_pad, e, d):
    rows_per_sub = n_pad // NS
    edges_per_tile = e // NW
    n_chunks = edges_per_tile // CH
    zrows = 16

    @functools.partial(
        pl.kernel,
        out_type=jax.ShapeDtypeStruct((NC * n_pad, d), jnp.float32),
        mesh=_mesh(),
        scratch_types=[
            pltpu.VMEM_SHARED((n_pad, d), jnp.float32),
            pltpu.VMEM((zrows, d), jnp.float32),
            pltpu.VMEM((CH,), jnp.int32),
            pltpu.VMEM((CH,), jnp.int32),
            pltpu.VMEM((CH, d), jnp.float32),
            pltpu.SemaphoreType.DMA,
        ],
    )
    def spmm(rows_hbm, cols_hbm, y_hbm, p_hbm, s_sh, zb, rows_v, cols_v, gbuf,
             sem):
        c = lax.axis_index("c")
        s = lax.axis_index("s")
        w = s * NC + c

        for r in range(zrows):
            for cc in range(d // LANES):
                zb[r, pl.ds(cc * LANES, LANES)] = jnp.zeros((LANES,), jnp.float32)

        def zero_loop(i, carry):
            pltpu.sync_copy(zb, s_sh.at[pl.ds(s * rows_per_sub + i * zrows, zrows)])
            return carry

        lax.fori_loop(0, rows_per_sub // zrows, zero_loop, None)
        plsc.subcore_barrier()

        def chunk(k, carry):
            base = w * edges_per_tile + k * CH
            pltpu.sync_copy(rows_hbm.at[pl.ds(base, CH)], rows_v)
            pltpu.sync_copy(cols_hbm.at[pl.ds(base, CH)], cols_v)
            pltpu.async_copy(y_hbm.at[cols_v], gbuf, sem).wait()
            pltpu.sync_copy(gbuf, s_sh.at[rows_v], add=True)
            return carry

        lax.fori_loop(0, n_chunks, chunk, None)
        plsc.subcore_barrier()

        pltpu.sync_copy(
            s_sh.at[pl.ds(s * rows_per_sub, rows_per_sub)],
            p_hbm.at[pl.ds(c * n_pad + s * rows_per_sub, rows_per_sub)],
        )

    return spmm


def _combine_body(x_ref, p0_ref, p1_ref, inv_ref, w_ref, b_ref, o_ref):
    inv = inv_ref[...]
    support = (p0_ref[...] + p1_ref[...]) * inv + (inv * inv) * x_ref[...]
    o_ref[...] = (
        lax.dot_general(
            support, w_ref[...], (((1,), (1,)), ((), ())),
            preferred_element_type=jnp.float32,
        )
        + b_ref[...]
    )


def _combine(x_pad, p0, p1, inv, W, b2, blk):
    n_pad, d = x_pad.shape
    grid = (n_pad // blk,)
    bs = pl.BlockSpec((blk, d), lambda i: (i, 0))
    return pl.pallas_call(
        _combine_body,
        grid=grid,
        in_specs=[
            bs, bs, bs,
            pl.BlockSpec((blk, 1), lambda i: (i, 0)),
            pl.BlockSpec((d, d), lambda i: (0, 0)),
            pl.BlockSpec((1, d), lambda i: (0, 0)),
        ],
        out_specs=bs,
        out_shape=jax.ShapeDtypeStruct((n_pad, d), jnp.float32),
    )(x_pad, p0, p1, inv, W, b2)


def kernel(x, edge_index, W, b):
    n, d = x.shape
    e = edge_index.shape[1]
    n_pad = ((n + 8 * NW - 1) // (8 * NW)) * (8 * NW)

    rows = edge_index[0].astype(jnp.int32)
    cols = edge_index[1].astype(jnp.int32)
    x_pad = jnp.pad(x, ((0, n_pad - n), (0, 0)))

    y_pad, inv = _make_deg_y(n_pad, e, d)(rows, x_pad)
    p = _make_spmm(n_pad, e, d)(rows, cols, y_pad)
    out_pad = _combine(
        x_pad, p[:n_pad], p[n_pad:], inv.reshape(-1, 1), W, b.reshape(1, -1),
        blk=1024,
    )
    return out_pad[:n]


# speedup vs baseline: 3.2942x; 2.4141x over previous
import functools

import jax
import jax.numpy as jnp
from jax import lax
from jax.experimental import pallas as pl
from jax.experimental.pallas import tpu as pltpu
from jax.experimental.pallas import tpu_sc as plsc

NC = 2
NS = 16
NW = NC * NS
LANES = 16
CH = 128


def _mesh():
    return plsc.VectorSubcoreMesh(
        core_axis_name="c", subcore_axis_name="s", num_cores=NC, num_subcores=NS
    )


def _make_deg_y(n_pad, e_pad, d):
    rows_per_sub = n_pad // NS
    rows_per_tile = n_pad // NW
    n_chunks_tot = e_pad // CH
    n_chunks = n_chunks_tot // NS

    @functools.partial(
        pl.kernel,
        out_type=[
            jax.ShapeDtypeStruct((n_pad, d), jnp.float32),
            jax.ShapeDtypeStruct((n_pad,), jnp.float32),
        ],
        mesh=_mesh(),
        scratch_types=[
            pltpu.VMEM_SHARED((n_pad,), jnp.float32),
            pltpu.VMEM((rows_per_sub,), jnp.float32),
            pltpu.VMEM((rows_per_sub,), jnp.float32),
            pltpu.VMEM((CH,), jnp.float32),
            pltpu.VMEM((n_chunks, CH), jnp.int32),
            pltpu.VMEM((rows_per_tile, d), jnp.float32),
        ],
    )
    def deg_y(rows_hbm, x_hbm, y_hbm, inv_hbm, deg_sh, buf_a, buf_b, ones_v,
              idx_v, xblk):
        c = lax.axis_index("c")
        s = lax.axis_index("s")

        pltpu.sync_copy(rows_hbm.at[pl.ds(s * n_chunks, n_chunks)], idx_v)

        def zero_loop(i, carry):
            buf_a[pl.ds(i * LANES, LANES)] = jnp.zeros((LANES,), jnp.float32)
            return carry

        lax.fori_loop(0, rows_per_sub // LANES, zero_loop, None)
        pltpu.sync_copy(buf_a, deg_sh.at[pl.ds(s * rows_per_sub, rows_per_sub)])

        def ones_loop(i, carry):
            ones_v[pl.ds(i * LANES, LANES)] = jnp.ones((LANES,), jnp.float32)
            return carry

        lax.fori_loop(0, CH // LANES, ones_loop, None)
        plsc.subcore_barrier()

        def acc(k, carry):
            pltpu.sync_copy(ones_v, deg_sh.at[idx_v.at[k]], add=True)
            return carry

        lax.fori_loop(0, n_chunks, acc, None)
        plsc.subcore_barrier()

        pltpu.sync_copy(deg_sh.at[pl.ds(s * rows_per_sub, rows_per_sub)], buf_a)

        def inv_loop(i, carry):
            v = buf_a[pl.ds(i * LANES, LANES)]
            buf_b[pl.ds(i * LANES, LANES)] = 1.0 / (v + 1.0)
            return carry

        lax.fori_loop(0, rows_per_sub // LANES, inv_loop, None)

        @pl.when(c == 0)
        def _():
            pltpu.sync_copy(buf_b, inv_hbm.at[pl.ds(s * rows_per_sub, rows_per_sub)])

        w = s * NC + c
        rowbase = w * rows_per_tile
        pltpu.sync_copy(x_hbm.at[pl.ds(rowbase, rows_per_tile)], xblk)

        def y_loop(g, carry):
            inv16 = buf_b[pl.ds(c * rows_per_tile + g * LANES, LANES)]
            for j in range(LANES):
                iv = inv16[j]
                r = g * LANES + j
                for cc in range(d // LANES):
                    xv = xblk[r, pl.ds(cc * LANES, LANES)]
                    xblk[r, pl.ds(cc * LANES, LANES)] = xv * iv
            return carry

        lax.fori_loop(0, rows_per_tile // LANES, y_loop, None)
        pltpu.sync_copy(xblk, y_hbm.at[pl.ds(rowbase, rows_per_tile)])

    return deg_y


def _make_spmm(n_pad, e_pad, d, ch):
    rows_per_sub = n_pad // NS
    chunks_per_tile = e_pad // ch // NW
    zrows = 16

    @functools.partial(
        pl.kernel,
        out_type=jax.ShapeDtypeStruct((NC * n_pad, d), jnp.float32),
        mesh=_mesh(),
        scratch_types=[
            pltpu.VMEM_SHARED((n_pad, d), jnp.float32),
            pltpu.VMEM((zrows, d), jnp.float32),
            pltpu.VMEM((ch,), jnp.int32),
            pltpu.VMEM((ch,), jnp.int32),
            pltpu.VMEM((ch,), jnp.int32),
            pltpu.VMEM((ch,), jnp.int32),
            pltpu.VMEM((ch, d), jnp.float32),
            pltpu.VMEM((ch, d), jnp.float32),
            pltpu.SemaphoreType.DMA,
            pltpu.SemaphoreType.DMA,
            pltpu.SemaphoreType.DMA,
            pltpu.SemaphoreType.DMA,
        ],
    )
    def spmm(rows_hbm, cols_hbm, y_hbm, p_hbm, s_sh, zb, r0, c0, r1, c1,
             g0, g1, sem0, sem1, isem0, isem1):
        c = lax.axis_index("c")
        s = lax.axis_index("s")
        w = s * NC + c
        base = w * chunks_per_tile * ch
        n_pairs = chunks_per_tile // 2

        def idx_start(k, rbuf, cbuf, isem):
            pltpu.async_copy(rows_hbm.at[pl.ds(base + k * ch, ch)], rbuf, isem)
            pltpu.async_copy(cols_hbm.at[pl.ds(base + k * ch, ch)], cbuf, isem)

        def idx_wait(rbuf, cbuf, isem):
            pltpu.make_async_copy(rows_hbm.at[pl.ds(base, ch)], rbuf, isem).wait()
            pltpu.make_async_copy(cols_hbm.at[pl.ds(base, ch)], cbuf, isem).wait()

        pltpu.sync_copy(rows_hbm.at[pl.ds(base, ch)], r0)
        pltpu.sync_copy(cols_hbm.at[pl.ds(base, ch)], c0)
        pltpu.async_copy(y_hbm.at[c0], g0, sem0)
        idx_start(1, r1, c1, isem1)

        for r in range(zrows):
            for cc in range(d // LANES):
                zb[r, pl.ds(cc * LANES, LANES)] = jnp.zeros((LANES,), jnp.float32)

        def zero_loop(i, carry):
            pltpu.sync_copy(zb, s_sh.at[pl.ds(s * rows_per_sub + i * zrows, zrows)])
            return carry

        lax.fori_loop(0, rows_per_sub // zrows, zero_loop, None)
        plsc.subcore_barrier()

        def pair(j, carry):
            k0 = 2 * j
            idx_wait(r1, c1, isem1)
            pltpu.async_copy(y_hbm.at[c1], g1, sem1)
            pltpu.make_async_copy(y_hbm.at[c0], g0, sem0).wait()
            pltpu.sync_copy(g0, s_sh.at[r0], add=True)

            @pl.when(j < n_pairs - 1)
            def _():
                idx_start(k0 + 2, r0, c0, isem0)

            pltpu.make_async_copy(y_hbm.at[c1], g1, sem1).wait()

            @pl.when(j < n_pairs - 1)
            def _():
                idx_wait(r0, c0, isem0)
                pltpu.async_copy(y_hbm.at[c0], g0, sem0)

            pltpu.sync_copy(g1, s_sh.at[r1], add=True)

            @pl.when(j < n_pairs - 1)
            def _():
                idx_start(k0 + 3, r1, c1, isem1)

            return carry

        lax.fori_loop(0, n_pairs, pair, None)
        plsc.subcore_barrier()

        pltpu.sync_copy(
            s_sh.at[pl.ds(s * rows_per_sub, rows_per_sub)],
            p_hbm.at[pl.ds(c * n_pad + s * rows_per_sub, rows_per_sub)],
        )

    return spmm


def _combine_body(x_ref, p0_ref, p1_ref, inv_ref, w_ref, b_ref, o_ref):
    inv = inv_ref[...]
    support = (p0_ref[...] + p1_ref[...]) * inv + (inv * inv) * x_ref[...]
    o_ref[...] = (
        lax.dot_general(
            support, w_ref[...], (((1,), (1,)), ((), ())),
            preferred_element_type=jnp.float32,
        )
        + b_ref[...]
    )


def _combine(x_pad, p0, p1, inv, W, b2, blk):
    n_pad, d = x_pad.shape
    grid = (n_pad // blk,)
    bs = pl.BlockSpec((blk, d), lambda i: (i, 0))
    return pl.pallas_call(
        _combine_body,
        grid=grid,
        in_specs=[
            bs, bs, bs,
            pl.BlockSpec((blk, 1), lambda i: (i, 0)),
            pl.BlockSpec((d, d), lambda i: (0, 0)),
            pl.BlockSpec((1, d), lambda i: (0, 0)),
        ],
        out_specs=bs,
        out_shape=jax.ShapeDtypeStruct((n_pad, d), jnp.float32),
    )(x_pad, p0, p1, inv, W, b2)


def kernel(x, edge_index, W, b):
    n, d = x.shape
    e = edge_index.shape[1]
    n_pad = ((n + 8 * NW - 1) // (8 * NW)) * (8 * NW)
    ept = -(-e // NW)
    ept = -(-ept // (2 * CH)) * (2 * CH)
    e_pad = ept * NW

    rows = edge_index[0].astype(jnp.int32)
    cols = edge_index[1].astype(jnp.int32)
    pad_idx = (n + jnp.arange(e_pad - e, dtype=jnp.int32) % (n_pad - n))
    rows_p = jnp.concatenate([rows, pad_idx])
    cols_p = jnp.concatenate([cols, pad_idx])
    x_pad = jnp.pad(x, ((0, n_pad - n), (0, 0)))

    bch = 128
    y_pad, inv = _make_deg_y(n_pad, e_pad, d)(rows_p.reshape(-1, CH), x_pad)
    p = _make_spmm(n_pad, e_pad, d, bch)(rows_p, cols_p, y_pad)
    out_pad = _combine(
        x_pad, p[:n_pad], p[n_pad:], inv.reshape(-1, 1), W, b.reshape(1, -1),
        blk=1024,
    )
    return out_pad[:n]
